# baseline (speedup 1.0000x reference)
import jax
import jax.numpy as jnp
from jax import lax
from jax.experimental import pallas as pl
from jax.experimental.pallas import tpu as pltpu
from jax.experimental.pallas import tpu_sc as plsc

_T = 8192
_D = 1024
_B = 4
_NW = 32
_CH = 16
_CE = _CH * _D
_TPW = _T // _NW
_NCH = _TPW // _CH
_STEPS = _NCH * _B
_U = 16


def _sc_body(x_hbm, pe_hbm, out_hbm,
             xb0, xb1, xb2, pb0, pb1,
             sx0, sx1, sx2, sp0, sp1, so0, so1, so2):
    xbufs = (xb0, xb1, xb2)
    pbufs = (pb0, pb1)
    sxs = (sx0, sx1, sx2)
    sps = (sp0, sp1)
    sos = (so0, so1, so2)

    c = lax.axis_index("c")
    s = lax.axis_index("s")
    wid = s * 2 + c
    pe_base = wid * _TPW * _D

    def pe_off(i):
        return pe_base + i * _CE

    def x_off(k):
        i, b = divmod(k, _B)
        return b * _T * _D + pe_base + i * _CE

    x_desc = [None] * _STEPS
    o_desc = [None] * _STEPS
    p_desc = [None] * _NCH

    def load_x(k):
        x_desc[k] = pltpu.async_copy(
            x_hbm.at[pl.ds(x_off(k), _CE)], xbufs[k % 3], sxs[k % 3])

    def load_pe(i):
        p_desc[i] = pltpu.async_copy(
            pe_hbm.at[pl.ds(pe_off(i), _CE)], pbufs[i % 2], sps[i % 2])

    load_pe(0)
    load_pe(1)
    load_x(0)
    load_x(1)

    for k in range(_STEPS):
        i, b = divmod(k, _B)
        x_desc[k].wait()
        if b == 0:
            p_desc[i].wait()
        xb = xbufs[k % 3]
        pb = pbufs[i % 2]

        @plsc.parallel_loop(0, _CE, step=16, unroll=_U)
        def vloop(o, xb=xb, pb=pb):
            xb[pl.ds(o, 16)] = xb[pl.ds(o, 16)] + pb[pl.ds(o, 16)]
        o_desc[k] = pltpu.async_copy(
            xb, out_hbm.at[pl.ds(x_off(k), _CE)], sos[k % 3])
        if k + 2 < _STEPS:
            if k - 1 >= 0:
                o_desc[k - 1].wait()
            load_x(k + 2)
        if b == _B - 1 and i + 2 < _NCH:
            load_pe(i + 2)

    o_desc[_STEPS - 3].wait()
    o_desc[_STEPS - 2].wait()
    o_desc[_STEPS - 1].wait()


def _sc_add(xf, pe):
    n = xf.shape[0]
    return pl.kernel(
        _sc_body,
        out_type=jax.ShapeDtypeStruct((n,), jnp.float32),
        mesh=plsc.VectorSubcoreMesh(core_axis_name="c", subcore_axis_name="s"),
        scratch_types=[
            pltpu.VMEM((_CE,), jnp.float32),
            pltpu.VMEM((_CE,), jnp.float32),
            pltpu.VMEM((_CE,), jnp.float32),
            pltpu.VMEM((_CE,), jnp.float32),
            pltpu.VMEM((_CE,), jnp.float32),
            pltpu.SemaphoreType.DMA,
            pltpu.SemaphoreType.DMA,
            pltpu.SemaphoreType.DMA,
            pltpu.SemaphoreType.DMA,
            pltpu.SemaphoreType.DMA,
            pltpu.SemaphoreType.DMA,
            pltpu.SemaphoreType.DMA,
            pltpu.SemaphoreType.DMA,
        ],
    )(xf, pe)


def kernel(x, pe):
    B, T, D = x.shape
    out = _sc_add(x.reshape(-1), pe.reshape(-1))
    return out.reshape(B, T, D)


# speedup vs baseline: 2.9746x; 2.9746x over previous
import jax
import jax.numpy as jnp
from jax import lax
from jax.experimental import pallas as pl
from jax.experimental.pallas import tpu as pltpu
from jax.experimental.pallas import tpu_sc as plsc

_T = 8192
_D = 1024
_B = 4
_NW = 32
_CH = 16
_TPW = _T // _NW
_NCH = _TPW // _CH


def _sc_body(x_hbm, pe_hbm, out_hbm,
             xb0, xb1, xb2, xb3, pb0, pb1,
             sx0, sx1, sx2, sx3, sp0, sp1, so0, so1, so2, so3):
    xbufs = (xb0, xb1, xb2, xb3)
    pbufs = (pb0, pb1)
    sxs = (sx0, sx1, sx2, sx3)
    sps = (sp0, sp1)
    sos = (so0, so1, so2, so3)

    c = lax.axis_index("c")
    s = lax.axis_index("s")
    wid = s * 2 + c
    row0 = wid * _TPW

    def pe_row(i):
        return row0 + i * _CH

    def start_load_x(i, b, bi):
        pltpu.async_copy(
            x_hbm.at[pl.ds(b * _T + pe_row(i), _CH)], xbufs[bi], sxs[bi])

    def start_load_pe(i, bi):
        pltpu.async_copy(pe_hbm.at[pl.ds(pe_row(i), _CH)], pbufs[bi], sps[bi])

    def wait_load_x(bi):
        pltpu.make_async_copy(
            x_hbm.at[pl.ds(row0, _CH)], xbufs[bi], sxs[bi]).wait()

    def wait_load_pe(bi):
        pltpu.make_async_copy(
            pe_hbm.at[pl.ds(row0, _CH)], pbufs[bi], sps[bi]).wait()

    def wait_store(bi):
        pltpu.make_async_copy(
            xbufs[bi], out_hbm.at[pl.ds(row0, _CH)], sos[bi]).wait()

    start_load_pe(0, 0)
    start_load_pe(1, 1)
    start_load_x(0, 0, 0)
    start_load_x(0, 1, 1)

    @pl.loop(0, _NCH, step=2)
    def chunk_body(iv):
        for ii in range(2):
            i = iv + ii
            for b in range(_B):
                wait_load_x(b)
                if b == 0:
                    wait_load_pe(ii)
                xb = xbufs[b]
                pb = pbufs[ii]

                @plsc.parallel_loop(0, _D, step=16)
                def vloop(o, xb=xb, pb=pb):
                    for r in range(_CH):
                        xb[r, pl.ds(o, 16)] = (
                            xb[r, pl.ds(o, 16)] + pb[r, pl.ds(o, 16)])

                pltpu.async_copy(
                    xb, out_hbm.at[pl.ds(b * _T + pe_row(i), _CH)], sos[b])

                b2 = (b + 2) % _B
                i2 = i + (1 if b >= 2 else 0)
                if ii == 0 and b <= 1:
                    @pl.when(iv >= 1)
                    def _():
                        wait_store(b2)
                else:
                    wait_store(b2)

                @pl.when(i2 < _NCH)
                def _():
                    start_load_x(i2, b2, b2)
                if b == _B - 1:
                    @pl.when(i + 2 < _NCH)
                    def _():
                        start_load_pe(i + 2, ii)

    wait_store(2)
    wait_store(3)


def _sc_add(xf, pe):
    n = xf.shape[0]
    return pl.kernel(
        _sc_body,
        out_type=jax.ShapeDtypeStruct((n, _D), jnp.float32),
        mesh=plsc.VectorSubcoreMesh(core_axis_name="c", subcore_axis_name="s"),
        compiler_params=pltpu.CompilerParams(use_tc_tiling_on_sc=True),
        scratch_types=[
            pltpu.VMEM((_CH, _D), jnp.float32),
            pltpu.VMEM((_CH, _D), jnp.float32),
            pltpu.VMEM((_CH, _D), jnp.float32),
            pltpu.VMEM((_CH, _D), jnp.float32),
            pltpu.VMEM((_CH, _D), jnp.float32),
            pltpu.VMEM((_CH, _D), jnp.float32),
            pltpu.SemaphoreType.DMA,
            pltpu.SemaphoreType.DMA,
            pltpu.SemaphoreType.DMA,
            pltpu.SemaphoreType.DMA,
            pltpu.SemaphoreType.DMA,
            pltpu.SemaphoreType.DMA,
            pltpu.SemaphoreType.DMA,
            pltpu.SemaphoreType.DMA,
            pltpu.SemaphoreType.DMA,
            pltpu.SemaphoreType.DMA,
        ],
    )(xf, pe)


def kernel(x, pe):
    B, T, D = x.shape
    out = _sc_add(x.reshape(B * T, D), pe)
    return out.reshape(B, T, D)
